# Initial kernel scaffold; baseline (speedup 1.0000x reference)
import jax
import jax.numpy as jnp
from jax.experimental import pallas as pl


def kernel(target_pressure, tau_p_axis, tau_vals, dead_p_axis, dead_vals):
    raise NotImplementedError("write your pallas kernel here")


# speedup vs baseline: 5.4787x; 5.4787x over previous
import jax
import jax.numpy as jnp
from jax import lax
from jax.experimental import pallas as pl
from jax.experimental.pallas import tpu as pltpu
from jax.experimental.pallas import tpu_sc as plsc

DT = 0.005
BUFFER_LEN = 22

NC = 2
NS = 16
LANES = 16
NW = NC * NS

N, C = 16384, 64
TOTAL = N * C
CHUNK = TOTAL // NW
UNROLL = 8
NVEC = CHUNK // LANES

NPTS = 6
LUT_PAD = 8


def _const(v, dtype=jnp.float32):
    return jnp.full((LANES,), v, dtype=dtype)


def _pam_body(p_hbm, lut_hbm, out_hbm, in_v, out_v, lut_v):
    wid = lax.axis_index("c") * NS + lax.axis_index("s")
    pltpu.sync_copy(lut_hbm, lut_v)
    pltpu.sync_copy(p_hbm.at[wid], in_v)

    def segments(xp_t, fp_t):
        f0 = lut_v[fp_t * NPTS]
        segs = []
        for j in range(NPTS - 1):
            x_lo = lut_v[xp_t * NPTS + j]
            x_hi = lut_v[xp_t * NPTS + j + 1]
            f_lo = lut_v[fp_t * NPTS + j]
            f_hi = lut_v[fp_t * NPTS + j + 1]
            width = x_hi - x_lo
            slope = (f_hi - f_lo) / (width + 1e-12)
            segs.append((x_lo, width, slope))
        return f0, segs

    tau_f0, tau_segs = segments(0, 1)
    dead_f0, dead_segs = segments(2, 3)

    zero = _const(0.0)
    one = _const(1.0)
    dt_v = _const(DT)
    buf_len = _const(float(BUFFER_LEN))
    d_max = _const(float(BUFFER_LEN - 2))
    last_slot = _const(BUFFER_LEN - 1, jnp.int32)
    zero_i = _const(0, jnp.int32)

    def interp(x, f0, segs):
        acc = f0
        for x_lo, width, slope in segs:
            c = jnp.minimum(jnp.maximum(x - x_lo, zero), width)
            acc = acc + slope * c
        return acc

    def body(i, carry):
        base = i * (LANES * UNROLL)
        for u in range(UNROLL):
            off = base + u * LANES
            x = in_v[pl.ds(off, LANES)]
            l_val = interp(x, dead_f0, dead_segs)
            tau = interp(x, tau_f0, tau_segs)
            d = jnp.minimum(jnp.maximum(l_val / dt_v, zero), d_max)
            r = jnp.where(d > zero, buf_len - d, zero)
            i0 = r.astype(jnp.int32)
            alpha = r - i0.astype(jnp.float32)
            w = (jnp.where(i0 == zero_i, one - alpha, zero)
                 + jnp.where(i0 == last_slot, alpha, zero))
            out_v[pl.ds(off, LANES)] = x * w * (dt_v / (tau + dt_v))
        return carry

    lax.fori_loop(0, NVEC // UNROLL, body, 0)
    pltpu.sync_copy(out_v, out_hbm.at[wid])


@jax.jit
def kernel(target_pressure, tau_p_axis, tau_vals, dead_p_axis, dead_vals):
    p2d = target_pressure.reshape(NW, CHUNK)
    lut = jnp.broadcast_to(
        jnp.stack((tau_p_axis, tau_vals, dead_p_axis, dead_vals))
        .reshape(4 * NPTS, 1), (4 * NPTS, LANES))

    sc_kernel = pl.kernel(
        _pam_body,
        out_type=jax.ShapeDtypeStruct((NW, CHUNK), jnp.float32),
        mesh=plsc.VectorSubcoreMesh(core_axis_name="c", subcore_axis_name="s"),
        scratch_types=[
            pltpu.VMEM((CHUNK,), jnp.float32),
            pltpu.VMEM((CHUNK,), jnp.float32),
            pltpu.VMEM((4 * NPTS, LANES), jnp.float32),
        ],
    )
    out = sc_kernel(p2d, lut)
    return out.reshape(N, C)
